# Initial kernel scaffold; baseline (speedup 1.0000x reference)
import jax
import jax.numpy as jnp
from jax.experimental import pallas as pl


def kernel(scores):
    raise NotImplementedError("write your pallas kernel here")


# speedup vs baseline: 9.5017x; 9.5017x over previous
import jax
import jax.numpy as jnp
from jax.experimental import pallas as pl

_B, _H, _N = 64, 16, 4096
_KS = (16, 64, 256, 1024)
_ROWS = _B * _H
_BLK = 128


def _gumbel_const():
    u = jax.random.uniform(jax.random.key(42), (_B, _H, _N), dtype=jnp.float32)
    g = -jnp.log(-jnp.log(u + 1e-20) + 1e-20)
    return g.reshape(_ROWS, _N)


_GUMBEL = _gumbel_const()


def _topk_mask_kernel(s_ref, g_ref, o_ref):
    p = s_ref[...] + g_ref[...]
    bits = jax.lax.bitcast_convert_type(p, jnp.int32)
    key = jnp.where(bits < 0, bits ^ jnp.int32(0x7FFFFFFF), bits)
    r = key.shape[0]
    int_min = jnp.int32(-(2**31))
    masks = []
    for k in _KS:
        t = jnp.full((r, 1), int_min, dtype=jnp.int32)
        cand = t & jnp.int32(0x7FFFFFFF)
        cnt = jnp.sum((key >= cand).astype(jnp.int32), axis=1, keepdims=True)
        t = jnp.where(cnt >= k, cand, t)
        for b in range(30, -1, -1):
            cand = t | jnp.int32(1 << b)
            cnt = jnp.sum((key >= cand).astype(jnp.int32), axis=1, keepdims=True)
            t = jnp.where(cnt >= k, cand, t)
        masks.append((key >= t).astype(jnp.float32))
    o_ref[...] = jnp.stack(masks, axis=1)


def kernel(scores):
    s2 = scores.reshape(_ROWS, _N)
    out = pl.pallas_call(
        _topk_mask_kernel,
        grid=(_ROWS // _BLK,),
        in_specs=[
            pl.BlockSpec((_BLK, _N), lambda i: (i, 0)),
            pl.BlockSpec((_BLK, _N), lambda i: (i, 0)),
        ],
        out_specs=pl.BlockSpec((_BLK, len(_KS), _N), lambda i: (i, 0, 0)),
        out_shape=jax.ShapeDtypeStruct((_ROWS, len(_KS), _N), jnp.float32),
    )(s2, _GUMBEL)
    return out.reshape(_B, _H, len(_KS), _N)
